# baseline (speedup 1.0000x reference)
import functools

import jax
import jax.numpy as jnp
from jax import lax
from jax.experimental import pallas as pl
from jax.experimental.pallas import tpu as pltpu
from jax.experimental.pallas import tpu_sc as plsc

B = 16384
L = 200
VOCAB_ROWS = 1000000
EMB_DIM = 10
DPAD = 16

NC = 2
NS = 16
LANES = 16
NW = NC * NS
BPW = B // NW
HALF = L // 2
NSTAGE = 2
RPS = BPW // NSTAGE
NBUF = 4

PCHUNK = 1600
NCHUNK = VOCAB_ROWS // PCHUNK
CPW = -(-NCHUNK // NW)
PUNROLL = 16

_MESH = plsc.VectorSubcoreMesh(core_axis_name="c", subcore_axis_name="s")
_SC_PARAMS = pltpu.CompilerParams(
    use_tc_tiling_on_sc=False, needs_layout_passes=False)


def _sc_pad(embT):
    @functools.partial(
        pl.kernel,
        mesh=_MESH,
        compiler_params=_SC_PARAMS,
        out_type=jax.ShapeDtypeStruct((VOCAB_ROWS, DPAD), jnp.float32),
        scratch_types=[
            pltpu.VMEM((2, EMB_DIM, PCHUNK), jnp.float32),
            pltpu.VMEM((2, PCHUNK, DPAD), jnp.float32),
            pltpu.SemaphoreType.DMA,
            pltpu.SemaphoreType.DMA,
            pltpu.SemaphoreType.DMA,
            pltpu.SemaphoreType.DMA,
        ],
    )
    def pad(emb_hbm, out_hbm, in_v, out_v, si0, si1, so0, so1):
        sin = (si0, si1)
        sout = (so0, so1)
        wid = lax.axis_index("s") * NC + lax.axis_index("c")
        lane = lax.iota(jnp.int32, LANES)
        dsel = jnp.where(lane < EMB_DIM, lane, 0)

        def chunk_of(k):
            return wid + k * NW

        def issue_in(slot, c):
            pltpu.async_copy(
                emb_hbm.at[:, pl.ds(c * PCHUNK, PCHUNK)], in_v.at[slot],
                sin[slot])

        def wait_in(slot):
            pltpu.make_async_copy(
                emb_hbm.at[:, pl.ds(0, PCHUNK)], in_v.at[slot],
                sin[slot]).wait()

        def issue_out(slot, c):
            pltpu.async_copy(
                out_v.at[slot], out_hbm.at[pl.ds(c * PCHUNK, PCHUNK)],
                sout[slot])

        def wait_out(slot):
            pltpu.make_async_copy(
                out_v.at[slot], out_hbm.at[pl.ds(0, PCHUNK)],
                sout[slot]).wait()

        def repitch(slot):
            src = in_v.at[slot]

            def step(i, carry):
                base = i * PUNROLL
                for j in range(PUNROLL):
                    col = jnp.zeros((LANES,), jnp.int32) + (base + j)
                    row = plsc.load_gather(src, [dsel, col])
                    out_v[slot, base + j] = row
                return carry

            lax.fori_loop(0, PCHUNK // PUNROLL, step, 0)

        @pl.when(chunk_of(0) < NCHUNK)
        def _():
            issue_in(0, chunk_of(0))
        for k in range(CPW):
            slot = k % 2
            if k + 1 < CPW:
                @pl.when(chunk_of(k + 1) < NCHUNK)
                def _():
                    issue_in(1 - slot, chunk_of(k + 1))
            if k >= 2:
                @pl.when(chunk_of(k - 2) < NCHUNK)
                def _():
                    wait_out(slot)

            @pl.when(chunk_of(k) < NCHUNK)
            def _():
                wait_in(slot)
                repitch(slot)
                issue_out(slot, chunk_of(k))
        for k in (CPW - 2, CPW - 1):
            @pl.when(chunk_of(k) < NCHUNK)
            def _():
                wait_out(k % 2)

    return pad(embT)


def _sc_pool(words2d, emb16):
    @functools.partial(
        pl.kernel,
        mesh=_MESH,
        compiler_params=_SC_PARAMS,
        out_type=jax.ShapeDtypeStruct((B, DPAD), jnp.float32),
        scratch_types=[
            pltpu.VMEM((RPS * 2, HALF), jnp.int32),
            pltpu.VMEM((NBUF, L, DPAD), jnp.float32),
            pltpu.VMEM((BPW, DPAD), jnp.float32),
            pltpu.SemaphoreType.DMA,
            pltpu.SemaphoreType.DMA,
            pltpu.SemaphoreType.DMA,
            pltpu.SemaphoreType.DMA,
        ],
    )
    def pool(words_hbm, emb_hbm, x_hbm, idx_v, rows_v, x_v, s0, s1, s2, s3):
        sems = (s0, s1, s2, s3)
        wid = lax.axis_index("s") * NC + lax.axis_index("c")
        row0 = wid * BPW

        def issue(buf, r):
            pltpu.async_copy(
                emb_hbm.at[idx_v.at[2 * r]],
                rows_v.at[buf, pl.ds(0, HALF)], sems[buf])
            pltpu.async_copy(
                emb_hbm.at[idx_v.at[2 * r + 1]],
                rows_v.at[buf, pl.ds(HALF, HALF)], sems[buf])

        def wait(buf):
            pltpu.make_async_copy(
                emb_hbm.at[pl.ds(0, L)], rows_v.at[buf], sems[buf]).wait()

        def reduce_into(buf, xr):
            accs = [rows_v[buf, j] for j in range(NBUF)]
            for base in range(NBUF, L, NBUF):
                for j in range(NBUF):
                    accs[j] = accs[j] + rows_v[buf, base + j]
            x_v[xr] = (accs[0] + accs[1]) + (accs[2] + accs[3])

        for s in range(NSTAGE):
            srow = s * RPS
            pltpu.sync_copy(
                words_hbm.at[pl.ds((row0 + srow) * 2, RPS * 2)], idx_v)
            for j in range(NBUF):
                issue(j, j)

            def body(i, carry):
                r = i * NBUF
                for j in range(NBUF):
                    wait(j)
                    reduce_into(j, srow + r + j)

                    @pl.when(r + j + NBUF < RPS)
                    def _():
                        issue(j, r + j + NBUF)
                return carry

            lax.fori_loop(0, RPS // NBUF, body, 0)

        pltpu.sync_copy(x_v, x_hbm.at[pl.ds(row0, BPW)])

    return pool(words2d, emb16)


def _tc_mlp(x, w1p, b1, w2, b2):
    def body(x_ref, w1_ref, b1_ref, w2_ref, b2_ref, o_ref):
        x = x_ref[...]
        col = lax.broadcasted_iota(jnp.int32, x.shape, 1)
        x = jnp.where(col < EMB_DIM, x, 0.0)
        h = jnp.dot(x, w1_ref[...],
                    preferred_element_type=jnp.float32) + b1_ref[...]
        h = jnp.maximum(h, 0.0)
        z = jnp.dot(h, w2_ref[...],
                    preferred_element_type=jnp.float32) + b2_ref[...]
        o_ref[...] = 1.0 / (1.0 + jnp.exp(-z))

    return pl.pallas_call(
        body,
        out_shape=jax.ShapeDtypeStruct((B, 1), jnp.float32),
    )(x, w1p, b1, w2, b2)


def kernel(words, emb, W1, b1, W2, b2):
    words2d = words.astype(jnp.int32).reshape(B * 2, HALF)
    emb16 = _sc_pad(emb.T)
    x = _sc_pool(words2d, emb16)
    w1p = jnp.zeros((DPAD, W1.shape[1]), jnp.float32).at[:EMB_DIM, :].set(W1)
    return _tc_mlp(x, w1p, b1.reshape(1, -1), W2, b2.reshape(1, 1))


# speedup vs baseline: 1.4303x; 1.4303x over previous
import functools

import jax
import jax.numpy as jnp
from jax import lax
from jax.experimental import pallas as pl
from jax.experimental.pallas import tpu as pltpu
from jax.experimental.pallas import tpu_sc as plsc

B = 16384
L = 200
VOCAB_ROWS = 1000000
EMB_DIM = 10
DPAD = 16

NC = 2
NS = 16
LANES = 16
NW = NC * NS
BPW = B // NW
HALF = L // 2
NSTAGE = 2
RPS = BPW // NSTAGE
NBUF = 8
NACC = 4

PCHUNK = 1600
NCHUNK = VOCAB_ROWS // PCHUNK
CPW = -(-NCHUNK // NW)
PUNROLL = 16

_MESH = plsc.VectorSubcoreMesh(core_axis_name="c", subcore_axis_name="s")
_SC_PARAMS = pltpu.CompilerParams(
    use_tc_tiling_on_sc=False, needs_layout_passes=False)


def _sc_pad(emb1d):
    @functools.partial(
        pl.kernel,
        mesh=_MESH,
        compiler_params=_SC_PARAMS,
        out_type=jax.ShapeDtypeStruct((VOCAB_ROWS, DPAD), jnp.float32),
        scratch_types=[
            pltpu.VMEM((2, PCHUNK * EMB_DIM + LANES), jnp.float32),
            pltpu.VMEM((2, PCHUNK, DPAD), jnp.float32),
            pltpu.SemaphoreType.DMA,
            pltpu.SemaphoreType.DMA,
            pltpu.SemaphoreType.DMA,
            pltpu.SemaphoreType.DMA,
        ],
    )
    def pad(emb_hbm, out_hbm, in_v, out_v, si0, si1, so0, so1):
        sin = (si0, si1)
        sout = (so0, so1)
        wid = lax.axis_index("s") * NC + lax.axis_index("c")
        lane = lax.iota(jnp.int32, LANES)
        cvec = [lane + EMB_DIM * j for j in range(PUNROLL)]

        def chunk_of(k):
            return wid + k * NW

        def issue_in(slot, c):
            pltpu.async_copy(
                emb_hbm.at[pl.ds(c * PCHUNK * EMB_DIM, PCHUNK * EMB_DIM)],
                in_v.at[slot, pl.ds(0, PCHUNK * EMB_DIM)], sin[slot])

        def wait_in(slot):
            pltpu.make_async_copy(
                emb_hbm.at[pl.ds(0, PCHUNK * EMB_DIM)],
                in_v.at[slot, pl.ds(0, PCHUNK * EMB_DIM)], sin[slot]).wait()

        def issue_out(slot, c):
            pltpu.async_copy(
                out_v.at[slot], out_hbm.at[pl.ds(c * PCHUNK, PCHUNK)],
                sout[slot])

        def wait_out(slot):
            pltpu.make_async_copy(
                out_v.at[slot], out_hbm.at[pl.ds(0, PCHUNK)],
                sout[slot]).wait()

        def repitch(slot):
            src = in_v.at[slot]

            def step(i, carry):
                base = i * PUNROLL
                gsplat = jnp.zeros((LANES,), jnp.int32) + base * EMB_DIM
                for j in range(PUNROLL):
                    row = plsc.load_gather(src, [gsplat + cvec[j]])
                    out_v[slot, base + j] = row
                return carry

            lax.fori_loop(0, PCHUNK // PUNROLL, step, 0)

        @pl.when(chunk_of(0) < NCHUNK)
        def _():
            issue_in(0, chunk_of(0))
        for k in range(CPW):
            slot = k % 2
            if k + 1 < CPW:
                @pl.when(chunk_of(k + 1) < NCHUNK)
                def _():
                    issue_in(1 - slot, chunk_of(k + 1))
            if k >= 2:
                @pl.when(chunk_of(k - 2) < NCHUNK)
                def _():
                    wait_out(slot)

            @pl.when(chunk_of(k) < NCHUNK)
            def _():
                wait_in(slot)
                repitch(slot)
                issue_out(slot, chunk_of(k))
        for k in (CPW - 2, CPW - 1):
            @pl.when(chunk_of(k) < NCHUNK)
            def _():
                wait_out(k % 2)

    return pad(emb1d)


def _sc_pool(words2d, emb16):
    @functools.partial(
        pl.kernel,
        mesh=_MESH,
        compiler_params=_SC_PARAMS,
        out_type=jax.ShapeDtypeStruct((B, DPAD), jnp.float32),
        scratch_types=[
            pltpu.VMEM((RPS * 2, HALF), jnp.int32),
            pltpu.VMEM((NBUF, L, DPAD), jnp.float32),
            pltpu.VMEM((BPW, DPAD), jnp.float32),
        ] + [pltpu.SemaphoreType.DMA] * NBUF,
    )
    def pool(words_hbm, emb_hbm, x_hbm, idx_v, rows_v, x_v, *sems):
        wid = lax.axis_index("s") * NC + lax.axis_index("c")
        row0 = wid * BPW

        def issue(buf, r):
            pltpu.async_copy(
                emb_hbm.at[idx_v.at[2 * r]],
                rows_v.at[buf, pl.ds(0, HALF)], sems[buf])
            pltpu.async_copy(
                emb_hbm.at[idx_v.at[2 * r + 1]],
                rows_v.at[buf, pl.ds(HALF, HALF)], sems[buf])

        def wait(buf):
            pltpu.make_async_copy(
                emb_hbm.at[pl.ds(0, L)], rows_v.at[buf], sems[buf]).wait()

        def reduce_into(buf, xr):
            accs = [rows_v[buf, j] for j in range(NACC)]
            for base in range(NACC, L, NACC):
                for j in range(NACC):
                    accs[j] = accs[j] + rows_v[buf, base + j]
            x_v[xr] = (accs[0] + accs[1]) + (accs[2] + accs[3])

        for s in range(NSTAGE):
            srow = s * RPS
            pltpu.sync_copy(
                words_hbm.at[pl.ds((row0 + srow) * 2, RPS * 2)], idx_v)
            for j in range(NBUF):
                issue(j, j)

            def body(i, carry):
                r = i * NBUF
                for j in range(NBUF):
                    wait(j)
                    reduce_into(j, srow + r + j)

                    @pl.when(r + j + NBUF < RPS)
                    def _():
                        issue(j, r + j + NBUF)
                return carry

            lax.fori_loop(0, RPS // NBUF, body, 0)

        pltpu.sync_copy(x_v, x_hbm.at[pl.ds(row0, BPW)])

    return pool(words2d, emb16)


def _tc_mlp(x, w1p, b1, w2, b2):
    def body(x_ref, w1_ref, b1_ref, w2_ref, b2_ref, o_ref):
        x = x_ref[...]
        col = lax.broadcasted_iota(jnp.int32, x.shape, 1)
        x = jnp.where(col < EMB_DIM, x, 0.0)
        h = jnp.dot(x, w1_ref[...],
                    preferred_element_type=jnp.float32) + b1_ref[...]
        h = jnp.maximum(h, 0.0)
        z = jnp.dot(h, w2_ref[...],
                    preferred_element_type=jnp.float32) + b2_ref[...]
        o_ref[...] = 1.0 / (1.0 + jnp.exp(-z))

    return pl.pallas_call(
        body,
        out_shape=jax.ShapeDtypeStruct((B, 1), jnp.float32),
    )(x, w1p, b1, w2, b2)


def kernel(words, emb, W1, b1, W2, b2):
    words2d = words.astype(jnp.int32).reshape(B * 2, HALF)
    emb16 = _sc_pad(emb.reshape(VOCAB_ROWS * EMB_DIM))
    x = _sc_pool(words2d, emb16)
    w1p = jnp.zeros((DPAD, W1.shape[1]), jnp.float32).at[:EMB_DIM, :].set(W1)
    return _tc_mlp(x, w1p, b1.reshape(1, -1), W2, b2.reshape(1, 1))


# speedup vs baseline: 1.5301x; 1.0698x over previous
import functools

import jax
import jax.numpy as jnp
from jax import lax
from jax.experimental import pallas as pl
from jax.experimental.pallas import tpu as pltpu
from jax.experimental.pallas import tpu_sc as plsc

B = 16384
L = 200
VOCAB_ROWS = 1000000
EMB_DIM = 10
DPAD = 16

NC = 2
NS = 16
LANES = 16
NW = NC * NS
BPW = B // NW
HALF = L // 2
NSTAGE = 2
RPS = BPW // NSTAGE
NBUF = 4
NACC = 4

PCHUNK = 1600
NCHUNK = VOCAB_ROWS // PCHUNK
CPW = -(-NCHUNK // NW)
PUNROLL = 16

_MESH = plsc.VectorSubcoreMesh(core_axis_name="c", subcore_axis_name="s")
_SC_PARAMS = pltpu.CompilerParams(
    use_tc_tiling_on_sc=False, needs_layout_passes=False)


def _sc_pad(emb1d):
    @functools.partial(
        pl.kernel,
        mesh=_MESH,
        compiler_params=_SC_PARAMS,
        out_type=jax.ShapeDtypeStruct((VOCAB_ROWS, DPAD), jnp.float32),
        scratch_types=[
            pltpu.VMEM((2, PCHUNK * EMB_DIM + LANES), jnp.float32),
            pltpu.VMEM((2, PCHUNK, DPAD), jnp.float32),
            pltpu.SemaphoreType.DMA,
            pltpu.SemaphoreType.DMA,
            pltpu.SemaphoreType.DMA,
            pltpu.SemaphoreType.DMA,
        ],
    )
    def pad(emb_hbm, out_hbm, in_v, out_v, si0, si1, so0, so1):
        sin = (si0, si1)
        sout = (so0, so1)
        wid = lax.axis_index("s") * NC + lax.axis_index("c")
        lane = lax.iota(jnp.int32, LANES)
        cvec = [lane + EMB_DIM * j for j in range(PUNROLL)]

        def chunk_of(k):
            return wid + k * NW

        def issue_in(slot, c):
            pltpu.async_copy(
                emb_hbm.at[pl.ds(c * PCHUNK * EMB_DIM, PCHUNK * EMB_DIM)],
                in_v.at[slot, pl.ds(0, PCHUNK * EMB_DIM)], sin[slot])

        def wait_in(slot):
            pltpu.make_async_copy(
                emb_hbm.at[pl.ds(0, PCHUNK * EMB_DIM)],
                in_v.at[slot, pl.ds(0, PCHUNK * EMB_DIM)], sin[slot]).wait()

        def issue_out(slot, c):
            pltpu.async_copy(
                out_v.at[slot], out_hbm.at[pl.ds(c * PCHUNK, PCHUNK)],
                sout[slot])

        def wait_out(slot):
            pltpu.make_async_copy(
                out_v.at[slot], out_hbm.at[pl.ds(0, PCHUNK)],
                sout[slot]).wait()

        def repitch(slot):
            src = in_v.at[slot]

            def step(i, carry):
                base = i * PUNROLL
                gsplat = jnp.zeros((LANES,), jnp.int32) + base * EMB_DIM
                for j in range(PUNROLL):
                    row = plsc.load_gather(src, [gsplat + cvec[j]])
                    out_v[slot, base + j] = row
                return carry

            lax.fori_loop(0, PCHUNK // PUNROLL, step, 0)

        @pl.when(chunk_of(0) < NCHUNK)
        def _():
            issue_in(0, chunk_of(0))
        for k in range(CPW):
            slot = k % 2
            if k + 1 < CPW:
                @pl.when(chunk_of(k + 1) < NCHUNK)
                def _():
                    issue_in(1 - slot, chunk_of(k + 1))
            if k >= 2:
                @pl.when(chunk_of(k - 2) < NCHUNK)
                def _():
                    wait_out(slot)

            @pl.when(chunk_of(k) < NCHUNK)
            def _():
                wait_in(slot)
                repitch(slot)
                issue_out(slot, chunk_of(k))
        for k in (CPW - 2, CPW - 1):
            @pl.when(chunk_of(k) < NCHUNK)
            def _():
                wait_out(k % 2)

    return pad(emb1d)


def _sc_pool(words2d, emb16):
    @functools.partial(
        pl.kernel,
        mesh=_MESH,
        compiler_params=_SC_PARAMS,
        out_type=jax.ShapeDtypeStruct((B, DPAD), jnp.float32),
        scratch_types=[
            pltpu.VMEM((RPS * 2, HALF), jnp.int32),
            pltpu.VMEM((NBUF, L, DPAD), jnp.float32),
            pltpu.VMEM((BPW, DPAD), jnp.float32),
        ] + [pltpu.SemaphoreType.DMA] * NBUF,
    )
    def pool(words_hbm, emb_hbm, x_hbm, idx_v, rows_v, x_v, *sems):
        wid = lax.axis_index("s") * NC + lax.axis_index("c")
        row0 = wid * BPW

        def issue(buf, r):
            pltpu.async_copy(
                emb_hbm.at[idx_v.at[2 * r]],
                rows_v.at[buf, pl.ds(0, HALF)], sems[buf])
            pltpu.async_copy(
                emb_hbm.at[idx_v.at[2 * r + 1]],
                rows_v.at[buf, pl.ds(HALF, HALF)], sems[buf])

        def wait(buf):
            pltpu.make_async_copy(
                emb_hbm.at[pl.ds(0, L)], rows_v.at[buf], sems[buf]).wait()

        def reduce_into(buf, xr):
            accs = [rows_v[buf, j] for j in range(NACC)]
            for base in range(NACC, L, NACC):
                for j in range(NACC):
                    accs[j] = accs[j] + rows_v[buf, base + j]
            x_v[xr] = (accs[0] + accs[1]) + (accs[2] + accs[3])

        for s in range(NSTAGE):
            srow = s * RPS
            pltpu.sync_copy(
                words_hbm.at[pl.ds((row0 + srow) * 2, RPS * 2)], idx_v)
            for j in range(NBUF):
                issue(j, j)

            def body(i, carry):
                r = i * NBUF
                for j in range(NBUF):
                    wait(j)
                    reduce_into(j, srow + r + j)

                    @pl.when(r + j + NBUF < RPS)
                    def _():
                        issue(j, r + j + NBUF)
                return carry

            lax.fori_loop(0, RPS // NBUF, body, 0)

        pltpu.sync_copy(x_v, x_hbm.at[pl.ds(row0, BPW)])

    return pool(words2d, emb16)


def _tc_mlp(x, w1p, b1, w2, b2):
    def body(x_ref, w1_ref, b1_ref, w2_ref, b2_ref, o_ref):
        x = x_ref[...]
        col = lax.broadcasted_iota(jnp.int32, x.shape, 1)
        x = jnp.where(col < EMB_DIM, x, 0.0)
        h = jnp.dot(x, w1_ref[...],
                    preferred_element_type=jnp.float32) + b1_ref[...]
        h = jnp.maximum(h, 0.0)
        z = jnp.dot(h, w2_ref[...],
                    preferred_element_type=jnp.float32) + b2_ref[...]
        o_ref[...] = 1.0 / (1.0 + jnp.exp(-z))

    return pl.pallas_call(
        body,
        out_shape=jax.ShapeDtypeStruct((B, 1), jnp.float32),
    )(x, w1p, b1, w2, b2)


def kernel(words, emb, W1, b1, W2, b2):
    words2d = words.astype(jnp.int32).reshape(B * 2, HALF)
    emb16 = _sc_pad(emb.reshape(VOCAB_ROWS * EMB_DIM))
    x = _sc_pool(words2d, emb16)
    w1p = jnp.zeros((DPAD, W1.shape[1]), jnp.float32).at[:EMB_DIM, :].set(W1)
    return _tc_mlp(x, w1p, b1.reshape(1, -1), W2, b2.reshape(1, 1))
